# baseline (speedup 1.0000x reference)
import functools

import jax
import jax.numpy as jnp
import numpy as np
from jax import lax
from jax.experimental import pallas as pl
from jax.experimental.pallas import tpu as pltpu
from jax.experimental.pallas import tpu_sc as plsc

N = 50000
NNZ = 800000
EDGE_NUM = 25000
IN = 64
H = 8
C = 8
Q = NNZ // 4

_NC = 2
_NS = 16
_NW = _NC * _NS


def _build_A(att4):
    eyeH = jnp.eye(H, dtype=att4.dtype)
    A = att4[:, :, :, None] * eyeH[None, :, None, :]
    return A.transpose(1, 2, 0, 3).reshape(H * C, 4 * H)


def _expander():
    e = np.zeros((H, 64), np.float32)
    for h in range(H):
        e[h, h * 8:(h + 1) * 8] = 1.0
    return jnp.asarray(e)


def _lrelu_exp(x):
    return jnp.exp(jnp.where(x >= 0, x, 0.2 * x))


def _proj_body(x_ref, w_ref, b_ref, ae_ref, eexp_ref, x0_ref, ua_ref, ub_ref):
    x0 = jnp.dot(x_ref[...], w_ref[...], preferred_element_type=jnp.float32)
    x0 = x0 + b_ref[...]
    x0_ref[...] = x0
    sv = jnp.dot(x0, ae_ref[...], preferred_element_type=jnp.float32)
    g = _lrelu_exp(sv)
    for k in range(4):
        g8 = g[:, k * H:(k + 1) * H]
        y = jnp.dot(g8, eexp_ref[...],
                    preferred_element_type=jnp.float32) * x0
        ua_ref[:, k, :] = jnp.concatenate([y[:, :32], g8], axis=1)
        ub_ref[:, k, :] = jnp.concatenate([y[:, 32:], g8], axis=1)


def _proj(X, Wt, b, A_e, Eexp):
    n = X.shape[0]
    BLK = 1000
    return pl.pallas_call(
        _proj_body,
        grid=(n // BLK,),
        in_specs=[
            pl.BlockSpec((BLK, IN), lambda i: (i, 0)),
            pl.BlockSpec((IN, 64), lambda i: (0, 0)),
            pl.BlockSpec((1, 64), lambda i: (0, 0)),
            pl.BlockSpec((64, 32), lambda i: (0, 0)),
            pl.BlockSpec((H, 64), lambda i: (0, 0)),
        ],
        out_specs=[
            pl.BlockSpec((BLK, 64), lambda i: (i, 0)),
            pl.BlockSpec((BLK, 4, 40), lambda i: (i, 0, 0)),
            pl.BlockSpec((BLK, 4, 40), lambda i: (i, 0, 0)),
        ],
        out_shape=[
            jax.ShapeDtypeStruct((n, 64), jnp.float32),
            jax.ShapeDtypeStruct((n, 4, 40), jnp.float32),
            jax.ShapeDtypeStruct((n, 4, 40), jnp.float32),
        ],
    )(X, Wt, b.reshape(1, 64), A_e, Eexp)


def _edge_tables_body(d_ref, av_ref, eexp_ref, w0_ref, w1_ref, w2_ref, w3_ref,
                      w4_ref):
    d = d_ref[...]
    Ua = d[0, 0] + d[1, 0]
    Ub = d[0, 1] + d[1, 1]
    dil = jnp.dot(Ua[:, 32:40] + 1e-16, eexp_ref[...],
                  preferred_element_type=jnp.float32)
    Xe = jnp.concatenate([Ua[:, :32], Ub[:, :32]], axis=1) / dil
    sv = jnp.dot(Xe, av_ref[...], preferred_element_type=jnp.float32)
    a = _lrelu_exp(sv)
    wrefs = [w0_ref, w1_ref, w2_ref, w3_ref]
    for k in range(4):
        a8 = a[:, k * H:(k + 1) * H]
        y = jnp.dot(a8, eexp_ref[...],
                    preferred_element_type=jnp.float32) * Xe
        for q in range(4):
            wrefs[q][:, k, :] = y[:, 16 * q:16 * q + 16]
        w4_ref[:, k, :] = jnp.concatenate([a8, a8], axis=1)


def _edge_tables(dumpA, A_v, Eexp):
    BLK = 1000
    n = EDGE_NUM
    outs = pl.pallas_call(
        _edge_tables_body,
        grid=(n // BLK,),
        in_specs=[
            pl.BlockSpec((2, 2, BLK, 40), lambda i: (0, 0, i, 0)),
            pl.BlockSpec((64, 32), lambda i: (0, 0)),
            pl.BlockSpec((H, 64), lambda i: (0, 0)),
        ],
        out_specs=[pl.BlockSpec((BLK, 4, 16), lambda i: (i, 0, 0))] * 5,
        out_shape=[jax.ShapeDtypeStruct((n, 4, 16), jnp.float32)] * 5,
    )(dumpA, A_v, Eexp)
    return [o.reshape(4 * n, 16) for o in outs]


def _final_body(d_ref, eexp_ref, out_ref):
    d = d_ref[...]
    den = (d[0, 4] + d[1, 4])[:, :H]
    dil = jnp.dot(den + 1e-16, eexp_ref[...],
                  preferred_element_type=jnp.float32)
    y = jnp.concatenate([d[0, q] + d[1, q] for q in range(4)], axis=1)
    out_ref[...] = jax.nn.relu(y / dil)


def _final(dumpB, Eexp):
    BLK = 1000
    return pl.pallas_call(
        _final_body,
        grid=(N // BLK,),
        in_specs=[
            pl.BlockSpec((2, 5, BLK, 16), lambda i: (0, 0, i, 0)),
            pl.BlockSpec((H, 64), lambda i: (0, 0)),
        ],
        out_specs=pl.BlockSpec((BLK, 64), lambda i: (i, 0)),
        out_shape=jax.ShapeDtypeStruct((N, 64), jnp.float32),
    )(dumpB, Eexp)


def _sc_gather_scatter(tables, gidx, sidx, R, B):
    P = len(tables)
    W = tables[0].shape[1]
    M = gidx.shape[0]
    per_w = M // _NW
    nch = per_w // B
    assert M % _NW == 0 and per_w % B == 0 and B % 8 == 0 and W % 8 == 0
    mesh = plsc.VectorSubcoreMesh(core_axis_name="c", subcore_axis_name="s")

    K2 = nch // 2

    @functools.partial(
        pl.kernel, mesh=mesh,
        out_type=jax.ShapeDtypeStruct((_NC, P, R, W), jnp.float32),
        compiler_params=pltpu.CompilerParams(use_tc_tiling_on_sc=False),
        scratch_types=[
            pltpu.VMEM((B,), jnp.int32),
            pltpu.VMEM((B,), jnp.int32),
            pltpu.VMEM((B,), jnp.int32),
            pltpu.VMEM((B,), jnp.int32),
            pltpu.VMEM((B, W), jnp.float32),
            pltpu.VMEM((B, W), jnp.float32),
            pltpu.VMEM_SHARED((R, W), jnp.float32),
            pltpu.SemaphoreType.DMA,
            pltpu.SemaphoreType.DMA,
            pltpu.SemaphoreType.DMA,
            pltpu.SemaphoreType.DMA,
        ],
    )
    def k(*refs):
        t_hbm = refs[:P]
        gidx_hbm, sidx_hbm, zeros_hbm, out_hbm = refs[P:P + 4]
        gi0, si0, gi1, si1, rows0, rows1, acc, sg0, sg1, ss0, ss1 = refs[P + 4:]
        cid = lax.axis_index("c")
        sid = lax.axis_index("s")
        base = (sid * _NC + cid) * per_w

        for p in range(P):
            @pl.when(sid == 0)
            def _():
                pltpu.sync_copy(zeros_hbm, acc)

            plsc.subcore_barrier()
            tp = t_hbm[p]

            pltpu.sync_copy(gidx_hbm.at[pl.ds(base, B)], gi0)
            pltpu.sync_copy(sidx_hbm.at[pl.ds(base, B)], si0)
            g0 = pltpu.async_copy(tp.at[gi0], rows0, sg0)

            def body(j2, carry):
                off1 = base + (2 * j2 + 1) * B
                pltpu.sync_copy(gidx_hbm.at[pl.ds(off1, B)], gi1)
                pltpu.sync_copy(sidx_hbm.at[pl.ds(off1, B)], si1)
                pltpu.async_copy(tp.at[gi1], rows1, sg1)
                pltpu.make_async_copy(tp.at[gi0], rows0, sg0).wait()
                s0 = pltpu.async_copy(rows0, acc.at[si0], ss0, add=True)
                pltpu.make_async_copy(tp.at[gi1], rows1, sg1).wait()
                s1 = pltpu.async_copy(rows1, acc.at[si1], ss1, add=True)
                s0.wait()

                @pl.when(2 * j2 + 2 < nch)
                def _():
                    off2 = base + (2 * j2 + 2) * B
                    pltpu.sync_copy(gidx_hbm.at[pl.ds(off2, B)], gi0)
                    pltpu.sync_copy(sidx_hbm.at[pl.ds(off2, B)], si0)
                    pltpu.async_copy(tp.at[gi0], rows0, sg0)

                s1.wait()
                return carry

            lax.fori_loop(0, K2, body, 0)
            if nch % 2 == 1:
                pltpu.make_async_copy(tp.at[gi0], rows0, sg0).wait()
                pltpu.sync_copy(rows0, acc.at[si0], add=True)
            plsc.subcore_barrier()

            @pl.when(sid == 0)
            def _():
                pltpu.sync_copy(acc, out_hbm.at[cid, p])

    return k(*tables, gidx, sidx, jnp.zeros((R, W), jnp.float32))


def kernel(X, W_w, W_b, att_v_user, att_v_poi, att_v_class, att_v_time,
           att_e_friend, att_e_visit, att_e_occurrence, att_e_self,
           vertex, edges,
           E_class_index_0, E_class_index_1, E_class_index_2, E_class_index_3, E_class_index,
           V_class_index_0, V_class_index_1, V_class_index_2, V_class_index_3, V_class_index):
    att_e = jnp.stack([att_e_friend[0], att_e_visit[0], att_e_occurrence[0], att_e_self[0]])
    att_v = jnp.stack([att_v_user[0], att_v_poi[0], att_v_class[0], att_v_time[0]])
    A_e = _build_A(att_e)
    A_v = _build_A(att_v)
    Eexp = _expander()

    X0, UEa, UEb = _proj(X, W_w.T, W_b, A_e, Eexp)
    UEa = UEa.reshape(4 * N, 40)
    UEb = UEb.reshape(4 * N, 40)

    rE = vertex * 4 + E_class_index[:, 0] // Q
    rV = edges * 4 + V_class_index[:, 0] // Q

    dumpA = _sc_gather_scatter([UEa, UEb], rE, edges, EDGE_NUM, B=200)
    WV = _edge_tables(dumpA, A_v, Eexp)
    dumpB = _sc_gather_scatter(WV, rV, vertex, N, B=200)
    return _final(dumpB, Eexp)


# speedup vs baseline: 1.2461x; 1.2461x over previous
import functools

import jax
import jax.numpy as jnp
import numpy as np
from jax import lax
from jax.experimental import pallas as pl
from jax.experimental.pallas import tpu as pltpu
from jax.experimental.pallas import tpu_sc as plsc

N = 50000
NNZ = 800000
EDGE_NUM = 25000
IN = 64
H = 8
C = 8
Q = NNZ // 4

_NC = 2
_NS = 16
_NW = _NC * _NS


def _build_A(att4):
    eyeH = jnp.eye(H, dtype=att4.dtype)
    A = att4[:, :, :, None] * eyeH[None, :, None, :]
    return A.transpose(1, 2, 0, 3).reshape(H * C, 4 * H)


def _expander():
    e = np.zeros((H, 64), np.float32)
    for h in range(H):
        e[h, h * 8:(h + 1) * 8] = 1.0
    return jnp.asarray(e)


def _lrelu_exp(x):
    return jnp.exp(jnp.where(x >= 0, x, 0.2 * x))


def _proj_body(x_ref, w_ref, b_ref, ae_ref, eexp_ref, x0_ref, ua_ref, ub_ref):
    x0 = jnp.dot(x_ref[...], w_ref[...], preferred_element_type=jnp.float32)
    x0 = x0 + b_ref[...]
    x0_ref[...] = x0
    sv = jnp.dot(x0, ae_ref[...], preferred_element_type=jnp.float32)
    g = _lrelu_exp(sv)
    for k in range(4):
        g8 = g[:, k * H:(k + 1) * H]
        y = jnp.dot(g8, eexp_ref[...],
                    preferred_element_type=jnp.float32) * x0
        ua_ref[:, k, :] = jnp.concatenate([y[:, :32], g8], axis=1)
        ub_ref[:, k, :] = jnp.concatenate([y[:, 32:], g8], axis=1)


def _proj(X, Wt, b, A_e, Eexp):
    n = X.shape[0]
    BLK = 1000
    return pl.pallas_call(
        _proj_body,
        grid=(n // BLK,),
        in_specs=[
            pl.BlockSpec((BLK, IN), lambda i: (i, 0)),
            pl.BlockSpec((IN, 64), lambda i: (0, 0)),
            pl.BlockSpec((1, 64), lambda i: (0, 0)),
            pl.BlockSpec((64, 32), lambda i: (0, 0)),
            pl.BlockSpec((H, 64), lambda i: (0, 0)),
        ],
        out_specs=[
            pl.BlockSpec((BLK, 64), lambda i: (i, 0)),
            pl.BlockSpec((BLK, 4, 40), lambda i: (i, 0, 0)),
            pl.BlockSpec((BLK, 4, 40), lambda i: (i, 0, 0)),
        ],
        out_shape=[
            jax.ShapeDtypeStruct((n, 64), jnp.float32),
            jax.ShapeDtypeStruct((n, 4, 40), jnp.float32),
            jax.ShapeDtypeStruct((n, 4, 40), jnp.float32),
        ],
    )(X, Wt, b.reshape(1, 64), A_e, Eexp)


def _edge_tables_body(d_ref, av_ref, eexp_ref, w0_ref, w1_ref, w2_ref, w3_ref,
                      w4_ref):
    d = d_ref[...]
    Ua = d[0, 0] + d[1, 0]
    Ub = d[0, 1] + d[1, 1]
    dil = jnp.dot(Ua[:, 32:40] + 1e-16, eexp_ref[...],
                  preferred_element_type=jnp.float32)
    Xe = jnp.concatenate([Ua[:, :32], Ub[:, :32]], axis=1) / dil
    sv = jnp.dot(Xe, av_ref[...], preferred_element_type=jnp.float32)
    a = _lrelu_exp(sv)
    wrefs = [w0_ref, w1_ref, w2_ref, w3_ref]
    for k in range(4):
        a8 = a[:, k * H:(k + 1) * H]
        y = jnp.dot(a8, eexp_ref[...],
                    preferred_element_type=jnp.float32) * Xe
        for q in range(4):
            wrefs[q][:, k, :] = y[:, 16 * q:16 * q + 16]
        w4_ref[:, k, :] = jnp.concatenate([a8, a8], axis=1)


def _edge_tables(dumpA, A_v, Eexp):
    BLK = 1000
    n = EDGE_NUM
    outs = pl.pallas_call(
        _edge_tables_body,
        grid=(n // BLK,),
        in_specs=[
            pl.BlockSpec((2, 2, BLK, 40), lambda i: (0, 0, i, 0)),
            pl.BlockSpec((64, 32), lambda i: (0, 0)),
            pl.BlockSpec((H, 64), lambda i: (0, 0)),
        ],
        out_specs=[pl.BlockSpec((BLK, 4, 16), lambda i: (i, 0, 0))] * 5,
        out_shape=[jax.ShapeDtypeStruct((n, 4, 16), jnp.float32)] * 5,
    )(dumpA, A_v, Eexp)
    return [o.reshape(4 * n, 16) for o in outs]


def _final_body(d_ref, eexp_ref, out_ref):
    d = d_ref[...]
    den = (d[0, 4] + d[1, 4])[:, :H]
    dil = jnp.dot(den + 1e-16, eexp_ref[...],
                  preferred_element_type=jnp.float32)
    y = jnp.concatenate([d[0, q] + d[1, q] for q in range(4)], axis=1)
    out_ref[...] = jax.nn.relu(y / dil)


def _final(dumpB, Eexp):
    BLK = 1000
    return pl.pallas_call(
        _final_body,
        grid=(N // BLK,),
        in_specs=[
            pl.BlockSpec((2, 5, BLK, 16), lambda i: (0, 0, i, 0)),
            pl.BlockSpec((H, 64), lambda i: (0, 0)),
        ],
        out_specs=pl.BlockSpec((BLK, 64), lambda i: (i, 0)),
        out_shape=jax.ShapeDtypeStruct((N, 64), jnp.float32),
    )(dumpB, Eexp)


def _sc_gather_scatter(tables, gidx, sidx, R, B):
    P = len(tables)
    W = tables[0].shape[1]
    M = gidx.shape[0]
    per_w = M // _NW
    nch = per_w // B
    assert M % _NW == 0 and per_w % B == 0 and B % 8 == 0 and W % 8 == 0
    mesh = plsc.VectorSubcoreMesh(core_axis_name="c", subcore_axis_name="s")

    @functools.partial(
        pl.kernel, mesh=mesh,
        out_type=jax.ShapeDtypeStruct((_NC, P, R, W), jnp.float32),
        compiler_params=pltpu.CompilerParams(use_tc_tiling_on_sc=False),
        scratch_types=[
            pltpu.VMEM((B,), jnp.int32),
            pltpu.VMEM((B,), jnp.int32),
            pltpu.VMEM((B, W), jnp.float32),
            pltpu.VMEM_SHARED((R, W), jnp.float32),
            pltpu.SemaphoreType.DMA,
        ],
    )
    def k(*refs):
        t_hbm = refs[:P]
        gidx_hbm, sidx_hbm, zeros_hbm, out_hbm = refs[P:P + 4]
        gi, si, rows, acc, sem = refs[P + 4:]
        cid = lax.axis_index("c")
        sid = lax.axis_index("s")
        base = (sid * _NC + cid) * per_w
        for p in range(P):
            @pl.when(sid == 0)
            def _():
                pltpu.sync_copy(zeros_hbm, acc)

            plsc.subcore_barrier()
            tp = t_hbm[p]

            def body(j, carry):
                off = base + j * B
                pltpu.sync_copy(gidx_hbm.at[pl.ds(off, B)], gi)
                pltpu.sync_copy(sidx_hbm.at[pl.ds(off, B)], si)
                pltpu.async_copy(tp.at[gi], rows, sem).wait()
                pltpu.sync_copy(rows, acc.at[si], add=True)
                return carry

            lax.fori_loop(0, nch, body, 0)
            plsc.subcore_barrier()

            @pl.when(sid == 0)
            def _():
                pltpu.sync_copy(acc, out_hbm.at[cid, p])

    return k(*tables, gidx, sidx, jnp.zeros((R, W), jnp.float32))


def kernel(X, W_w, W_b, att_v_user, att_v_poi, att_v_class, att_v_time,
           att_e_friend, att_e_visit, att_e_occurrence, att_e_self,
           vertex, edges,
           E_class_index_0, E_class_index_1, E_class_index_2, E_class_index_3, E_class_index,
           V_class_index_0, V_class_index_1, V_class_index_2, V_class_index_3, V_class_index):
    att_e = jnp.stack([att_e_friend[0], att_e_visit[0], att_e_occurrence[0], att_e_self[0]])
    att_v = jnp.stack([att_v_user[0], att_v_poi[0], att_v_class[0], att_v_time[0]])
    A_e = _build_A(att_e)
    A_v = _build_A(att_v)
    Eexp = _expander()

    X0, UEa, UEb = _proj(X, W_w.T, W_b, A_e, Eexp)
    UEa = UEa.reshape(4 * N, 40)
    UEb = UEb.reshape(4 * N, 40)

    rE = vertex * 4 + E_class_index[:, 0] // Q
    rV = edges * 4 + V_class_index[:, 0] // Q

    dumpA = _sc_gather_scatter([UEa, UEb], rE, edges, EDGE_NUM, B=1000)
    WV = _edge_tables(dumpA, A_v, Eexp)
    dumpB = _sc_gather_scatter(WV, rV, vertex, N, B=1000)
    return _final(dumpB, Eexp)
